# Initial kernel scaffold; baseline (speedup 1.0000x reference)
import jax
import jax.numpy as jnp
from jax.experimental import pallas as pl


def kernel(x, codebook):
    raise NotImplementedError("write your pallas kernel here")


# speedup vs baseline: 1.3458x; 1.3458x over previous
import functools

import jax
import jax.numpy as jnp
from jax import lax
from jax.experimental import pallas as pl
from jax.experimental.pallas import tpu as pltpu
from jax.experimental.pallas import tpu_sc as plsc

NUM_CODES = 1024
DIM = 64
TOKENS = 64 * 576
TOK_BLOCK = 1024
N_BLOCKS = TOKENS // TOK_BLOCK

SC_CORES = 2
SC_SUBCORES = 16
SC_WORKERS = SC_CORES * SC_SUBCORES
TOK_PER_W = TOKENS // SC_WORKERS
IDX_CHUNK = 128
CHUNKS_PER_W = TOK_PER_W // IDX_CHUNK


def _argmin_body(x_ref, cb_ref, cn_ref, idx_ref, loss_ref):
    i = pl.program_id(0)
    x = x_ref[...]
    cb = cb_ref[...]
    scores = lax.dot_general(
        x, cb, (((1,), (1,)), ((), ())), preferred_element_type=jnp.float32)
    dist = cn_ref[...] - 2.0 * scores
    m = jnp.min(dist, axis=1, keepdims=True)
    code_iota = lax.broadcasted_iota(jnp.int32, dist.shape, 1)
    idx_ref[...] = jnp.min(
        jnp.where(dist == m, code_iota, NUM_CODES), axis=1).astype(jnp.int32)
    xsq = jnp.sum(x * x, axis=1)
    blk_loss = jnp.sum(m[:, 0] + xsq)

    @pl.when(i == 0)
    def _():
        loss_ref[0, 0] = 0.0

    loss_ref[0, 0] += blk_loss


def _tc_argmin(x2d, codebook, cnorm):
    return pl.pallas_call(
        _argmin_body,
        grid=(N_BLOCKS,),
        in_specs=[
            pl.BlockSpec((TOK_BLOCK, DIM), lambda i: (i, 0)),
            pl.BlockSpec((NUM_CODES, DIM), lambda i: (0, 0)),
            pl.BlockSpec((1, NUM_CODES), lambda i: (0, 0)),
        ],
        out_specs=[
            pl.BlockSpec((TOK_BLOCK,), lambda i: (i,)),
            pl.BlockSpec(memory_space=pltpu.SMEM, block_shape=(1, 1),
                         index_map=lambda i: (0, 0)),
        ],
        out_shape=[
            jax.ShapeDtypeStruct((TOKENS,), jnp.int32),
            jax.ShapeDtypeStruct((1, 1), jnp.float32),
        ],
    )(x2d, codebook, cnorm)


@functools.cache
def _sc_gather_fn():
    mesh = plsc.VectorSubcoreMesh(core_axis_name="c", subcore_axis_name="s")

    @functools.partial(
        pl.kernel,
        mesh=mesh,
        out_type=jax.ShapeDtypeStruct((TOKENS, DIM), jnp.float32),
        scratch_types=[
            pltpu.VMEM((CHUNKS_PER_W, IDX_CHUNK), jnp.int32),
            pltpu.VMEM((TOK_PER_W, DIM), jnp.float32),
            pltpu.SemaphoreType.DMA,
        ],
        compiler_params=pltpu.CompilerParams(use_tc_tiling_on_sc=False),
    )
    def _sc_gather(cb_hbm, idx_hbm, out_hbm, idx_v, rows_v, sem):
        wid = lax.axis_index("s") * SC_CORES + lax.axis_index("c")
        pltpu.sync_copy(idx_hbm.at[wid], idx_v)
        for j in range(CHUNKS_PER_W):
            pltpu.async_copy(cb_hbm.at[idx_v.at[j]],
                             rows_v.at[pl.ds(j * IDX_CHUNK, IDX_CHUNK)], sem)
        for j in range(CHUNKS_PER_W):
            pltpu.make_async_copy(cb_hbm.at[idx_v.at[j]],
                                  rows_v.at[pl.ds(j * IDX_CHUNK, IDX_CHUNK)],
                                  sem).wait()
        pltpu.sync_copy(rows_v, out_hbm.at[pl.ds(wid * TOK_PER_W, TOK_PER_W)])

    return _sc_gather


def kernel(x, codebook):
    x2d = x.reshape(TOKENS, DIM)
    cnorm = jnp.sum(codebook ** 2, axis=1)[None, :]
    idx, loss_sum = _tc_argmin(x2d, codebook, cnorm)
    quantized = _sc_gather_fn()(codebook,
                                idx.reshape(SC_WORKERS, CHUNKS_PER_W,
                                            IDX_CHUNK))
    loss = 0.25 * loss_sum[0, 0] / (TOKENS * DIM)
    return quantized.reshape(x.shape), loss


# speedup vs baseline: 1.5820x; 1.1756x over previous
import functools

import jax
import jax.numpy as jnp
from jax import lax
from jax.experimental import pallas as pl
from jax.experimental.pallas import tpu as pltpu
from jax.experimental.pallas import tpu_sc as plsc

NUM_CODES = 1024
DIM = 64
TOKENS = 64 * 576
TOK_BLOCK = 1024
N_BLOCKS = TOKENS // TOK_BLOCK

SC_CORES = 2
SC_SUBCORES = 16
SC_WORKERS = SC_CORES * SC_SUBCORES
TOK_PER_W = TOKENS // SC_WORKERS
IDX_CHUNK = 128
CHUNKS_PER_W = TOK_PER_W // IDX_CHUNK


CODE_CHUNK = 128
N_CHUNKS = NUM_CODES // CODE_CHUNK


def _argmin_body(x_ref, cb_ref, cn_ref, idx_ref, loss_ref):
    i = pl.program_id(0)
    x = x_ref[...]
    lane = lax.broadcasted_iota(
        jnp.int32, (TOK_BLOCK, CODE_CHUNK), 1).astype(jnp.float32)
    best = jnp.full((TOK_BLOCK, CODE_CHUNK), jnp.inf, jnp.float32)
    bestidx = jnp.zeros((TOK_BLOCK, CODE_CHUNK), jnp.float32)
    for c in range(N_CHUNKS):
        cb_c = cb_ref[pl.ds(c * CODE_CHUNK, CODE_CHUNK), :]
        s = lax.dot_general(x, cb_c, (((1,), (1,)), ((), ())),
                            preferred_element_type=jnp.float32)
        dist = cn_ref[0, pl.ds(c * CODE_CHUNK, CODE_CHUNK)][None, :] + s
        upd = dist < best
        bestidx = jnp.where(upd, lane + float(c * CODE_CHUNK), bestidx)
        best = jnp.minimum(best, dist)
    m = jnp.min(best, axis=1, keepdims=True)
    idx_ref[...] = jnp.min(
        jnp.where(best == m, bestidx, float(NUM_CODES)),
        axis=1).astype(jnp.int32)
    xsq = jnp.sum(x * x, axis=1)
    blk_loss = jnp.sum(m[:, 0] + xsq)

    @pl.when(i == 0)
    def _():
        loss_ref[0, 0] = 0.0

    loss_ref[0, 0] += blk_loss


def _tc_argmin(x2d, codebook, cnorm):
    return pl.pallas_call(
        _argmin_body,
        grid=(N_BLOCKS,),
        in_specs=[
            pl.BlockSpec((TOK_BLOCK, DIM), lambda i: (i, 0)),
            pl.BlockSpec((NUM_CODES, DIM), lambda i: (0, 0)),
            pl.BlockSpec((1, NUM_CODES), lambda i: (0, 0)),
        ],
        out_specs=[
            pl.BlockSpec((TOK_BLOCK,), lambda i: (i,)),
            pl.BlockSpec(memory_space=pltpu.SMEM, block_shape=(1, 1),
                         index_map=lambda i: (0, 0)),
        ],
        out_shape=[
            jax.ShapeDtypeStruct((TOKENS,), jnp.int32),
            jax.ShapeDtypeStruct((1, 1), jnp.float32),
        ],
    )(x2d, codebook, cnorm)


@functools.cache
def _sc_gather_fn():
    mesh = plsc.VectorSubcoreMesh(core_axis_name="c", subcore_axis_name="s")

    @functools.partial(
        pl.kernel,
        mesh=mesh,
        out_type=jax.ShapeDtypeStruct((TOKENS, DIM), jnp.float32),
        scratch_types=[
            pltpu.VMEM((CHUNKS_PER_W, IDX_CHUNK), jnp.int32),
            pltpu.VMEM((TOK_PER_W, DIM), jnp.float32),
            pltpu.SemaphoreType.DMA,
        ],
        compiler_params=pltpu.CompilerParams(use_tc_tiling_on_sc=False),
    )
    def _sc_gather(cb_hbm, idx_hbm, out_hbm, idx_v, rows_v, sem):
        wid = lax.axis_index("s") * SC_CORES + lax.axis_index("c")
        pltpu.sync_copy(idx_hbm.at[wid], idx_v)
        for j in range(CHUNKS_PER_W):
            pltpu.async_copy(cb_hbm.at[idx_v.at[j]],
                             rows_v.at[pl.ds(j * IDX_CHUNK, IDX_CHUNK)], sem)
        for j in range(CHUNKS_PER_W):
            pltpu.make_async_copy(cb_hbm.at[idx_v.at[j]],
                                  rows_v.at[pl.ds(j * IDX_CHUNK, IDX_CHUNK)],
                                  sem).wait()
        pltpu.sync_copy(rows_v, out_hbm.at[pl.ds(wid * TOK_PER_W, TOK_PER_W)])

    return _sc_gather


def kernel(x, codebook):
    x2d = x.reshape(TOKENS, DIM)
    cnorm = jnp.sum(codebook ** 2, axis=1)[None, :]
    idx, loss_sum = _tc_argmin(x2d, -2.0 * codebook, cnorm)
    quantized = _sc_gather_fn()(codebook,
                                idx.reshape(SC_WORKERS, CHUNKS_PER_W,
                                            IDX_CHUNK))
    loss = 0.25 * loss_sum[0, 0] / (TOKENS * DIM)
    return quantized.reshape(x.shape), loss
